# baseline (speedup 1.0000x reference)
import functools

import jax
import jax.numpy as jnp
from jax import lax
from jax.experimental import pallas as pl
from jax.experimental.pallas import tpu as pltpu
from jax.experimental.pallas import tpu_sc as plsc

N = 10000
E = 320000
D = 128
ROWB = 2000
GRID = N // ROWB

NC, NS = 2, 16
K = 2048
NCHUNK = 157
E_PAD = K * NCHUNK
SENTINEL = 1 << 20

R0 = 313
R1 = 312
HALF0 = NS * R0
ACC_ROWS = 5024
EPT = E_PAD // NS
CH = 80
STAGE = 2160


def _layer_body(agg_ref, h_ref, wrel_ref, brel_ref, wroot_ref, o_ref):
    o_ref[...] = (
        jnp.dot(agg_ref[...], wrel_ref[...],
                preferred_element_type=jnp.float32)
        + brel_ref[...]
        + jnp.dot(h_ref[...], wroot_ref[...],
                  preferred_element_type=jnp.float32))


def _tc_layer(agg, h, wrel, brel, wroot):
    return pl.pallas_call(
        _layer_body,
        grid=(GRID,),
        in_specs=[
            pl.BlockSpec((ROWB, D), lambda i: (i, 0)),
            pl.BlockSpec((ROWB, D), lambda i: (i, 0)),
            pl.BlockSpec((D, D), lambda i: (0, 0)),
            pl.BlockSpec((1, D), lambda i: (0, 0)),
            pl.BlockSpec((D, D), lambda i: (0, 0)),
        ],
        out_specs=pl.BlockSpec((ROWB, D), lambda i: (i, 0)),
        out_shape=jax.ShapeDtypeStruct((N, D), jnp.float32),
    )(agg, h, wrel, brel.reshape(1, D), wroot)


def _mlp_body(h_ref, wm0_ref, bm0_ref, wm1_ref, bm1_ref, o_ref):
    t = jnp.dot(h_ref[...], wm0_ref[...], preferred_element_type=jnp.float32)
    t = jnp.maximum(t + bm0_ref[...], 0.0)
    o = jnp.dot(t, wm1_ref[...], preferred_element_type=jnp.float32)
    o_ref[...] = jax.nn.sigmoid(o + bm1_ref[...])


def _tc_mlp(h, wm0, bm0, wm1, bm1):
    return pl.pallas_call(
        _mlp_body,
        grid=(GRID,),
        in_specs=[
            pl.BlockSpec((ROWB, D), lambda i: (i, 0)),
            pl.BlockSpec((D, D), lambda i: (0, 0)),
            pl.BlockSpec((1, D), lambda i: (0, 0)),
            pl.BlockSpec((D, 1), lambda i: (0, 0)),
            pl.BlockSpec((1, 1), lambda i: (0, 0)),
        ],
        out_specs=pl.BlockSpec((ROWB, 1), lambda i: (i, 0)),
        out_shape=jax.ShapeDtypeStruct((N, 1), jnp.float32),
    )(h, wm0, bm0.reshape(1, D), wm1, bm1.reshape(1, 1))


@functools.partial(
    pl.kernel,
    out_type=jax.ShapeDtypeStruct((N, D), jnp.float32),
    mesh=plsc.VectorSubcoreMesh(core_axis_name="c", subcore_axis_name="s"),
    compiler_params=pltpu.CompilerParams(needs_layout_passes=False),
    scratch_types=[
        pltpu.VMEM((K,), jnp.int32),
        pltpu.VMEM((K,), jnp.int32),
        pltpu.VMEM((STAGE,), jnp.int32),
        pltpu.VMEM((STAGE,), jnp.int32),
        pltpu.VMEM((CH,), jnp.int32),
        pltpu.VMEM((CH,), jnp.int32),
        pltpu.VMEM((CH, D), jnp.float32),
        pltpu.VMEM_SHARED((ACC_ROWS, D), jnp.float32),
        pltpu.VMEM_SHARED((E_PAD,), jnp.int32),
        pltpu.VMEM_SHARED((E_PAD,), jnp.int32),
        pltpu.SemaphoreType.DMA,
    ],
)
def _sc_agg(h_hbm, src_hbm, dst_hbm, zeros_hbm, out_hbm,
            dchunk, schunk, stage_d, stage_s, flush_d, flush_s, rows,
            acc, dst_sp, src_sp, sem):
    c = lax.axis_index("c")
    s = lax.axis_index("s")
    base = c * HALF0
    rpt = jnp.where(c == 0, R0, R1)
    lo = base + s * rpt
    hi = lo + rpt
    owned = jnp.where(c == 0, HALF0, NS * R1)
    dummy = owned + s
    lov = jnp.full((16,), lo, jnp.int32)
    hiv = jnp.full((16,), hi, jnp.int32)
    basev = jnp.full((16,), base, jnp.int32)
    dummyv = jnp.full((16,), dummy, jnp.int32)
    sentv = jnp.full((16,), SENTINEL, jnp.int32)

    @pl.when(s < 15)
    def _():
        pltpu.sync_copy(zeros_hbm.at[pl.ds(s * 320, 320)],
                        acc.at[pl.ds(s * 320, 320)])

    @pl.when(s == 15)
    def _():
        pltpu.sync_copy(zeros_hbm.at[pl.ds(4800, 224)],
                        acc.at[pl.ds(4800, 224)])

    pltpu.sync_copy(dst_hbm.at[pl.ds(s * EPT, EPT)],
                    dst_sp.at[pl.ds(s * EPT, EPT)])
    pltpu.sync_copy(src_hbm.at[pl.ds(s * EPT, EPT)],
                    src_sp.at[pl.ds(s * EPT, EPT)])
    plsc.subcore_barrier()

    def flush_body(fo):
        for kk in range(CH // 16):
            flush_d[pl.ds(kk * 16, 16)] = stage_d[pl.ds(fo + kk * 16, 16)]
            flush_s[pl.ds(kk * 16, 16)] = stage_s[pl.ds(fo + kk * 16, 16)]
        pltpu.async_copy(h_hbm.at[flush_s], rows, sem).wait()
        pltpu.sync_copy(rows, acc.at[flush_d], add=True)

    def scan_chunk(k, ptr):
        pltpu.sync_copy(dst_sp.at[pl.ds(k * K, K)], dchunk)
        pltpu.sync_copy(src_sp.at[pl.ds(k * K, K)], schunk)

        def scan_vec(i, ptr):
            dvec = dchunk[pl.ds(i * 16, 16)]
            fake = dvec >= sentv
            m = jnp.logical_or(
                jnp.logical_and(dvec >= lov, dvec < hiv), fake)
            cnt = jnp.sum(m.astype(jnp.int32))

            @pl.when(cnt > 0)
            def _():
                svec = schunk[pl.ds(i * 16, 16)]
                dl = jnp.where(fake, dummyv, dvec - basev)
                plsc.store_compressed(stage_d.at[pl.ds(ptr, 16)], dl, mask=m)
                plsc.store_compressed(stage_s.at[pl.ds(ptr, 16)], svec, mask=m)

            return ptr + cnt

        ptr = lax.fori_loop(0, K // 16, scan_vec, ptr)

        def flush_cond(fo):
            return fo + CH <= ptr

        def flush_step(fo):
            flush_body(fo)
            return fo + CH

        fo = lax.while_loop(flush_cond, flush_step, jnp.int32(0))

        for kk in range(CH // 16):
            flush_d[pl.ds(kk * 16, 16)] = stage_d[pl.ds(fo + kk * 16, 16)]
            flush_s[pl.ds(kk * 16, 16)] = stage_s[pl.ds(fo + kk * 16, 16)]
        for kk in range(CH // 16):
            stage_d[pl.ds(kk * 16, 16)] = flush_d[pl.ds(kk * 16, 16)]
            stage_s[pl.ds(kk * 16, 16)] = flush_s[pl.ds(kk * 16, 16)]
        return ptr - fo

    lax.fori_loop(0, NCHUNK, scan_chunk, jnp.int32(0))
    plsc.subcore_barrier()

    @pl.when(s < 15)
    def _():
        pltpu.sync_copy(acc.at[pl.ds(s * 320, 320)],
                        out_hbm.at[pl.ds(base + s * 320, 320)])

    @pl.when(jnp.logical_and(s == 15, c == 0))
    def _():
        pltpu.sync_copy(acc.at[pl.ds(4800, 208)],
                        out_hbm.at[pl.ds(4800, 208)])

    @pl.when(jnp.logical_and(s == 15, c == 1))
    def _():
        pltpu.sync_copy(acc.at[pl.ds(4800, 192)],
                        out_hbm.at[pl.ds(HALF0 + 4800, 192)])


def kernel(x, edge_index, Wrel0, brel0, Wroot0, Wrel1, brel1, Wroot1,
           Wrel2, brel2, Wroot2, Wm0, bm0, Wm1, bm1):
    src = edge_index[0].astype(jnp.int32)
    dst = edge_index[1].astype(jnp.int32)
    pad = E_PAD - E
    src_p = jnp.concatenate([src, jnp.zeros((pad,), jnp.int32)])
    dst_p = jnp.concatenate([dst, jnp.full((pad,), SENTINEL, jnp.int32)])
    zeros = jnp.zeros((ACC_ROWS, D), jnp.float32)

    h = x
    for wrel, brel, wroot in ((Wrel0, brel0, Wroot0),
                              (Wrel1, brel1, Wroot1),
                              (Wrel2, brel2, Wroot2)):
        agg = _sc_agg(h, src_p, dst_p, zeros)
        h = _tc_layer(agg, h, wrel, brel, wroot)
    return _tc_mlp(h, Wm0, bm0, Wm1, bm1)


# speedup vs baseline: 1.2000x; 1.2000x over previous
import functools

import jax
import jax.numpy as jnp
from jax import lax
from jax.experimental import pallas as pl
from jax.experimental.pallas import tpu as pltpu
from jax.experimental.pallas import tpu_sc as plsc

N = 10000
E = 320000
D = 128
ROWB = 2000
GRID = N // ROWB

NC, NS = 2, 16
K = 2048
NCHUNK = 157
E_PAD = K * NCHUNK
SENTINEL = 1 << 20

R0 = 313
R1 = 312
HALF0 = NS * R0
ACC_ROWS = 5024
EPT = E_PAD // NS
CH = 128
STAGE = 2304


def _layer_body(agg_ref, h_ref, wrel_ref, brel_ref, wroot_ref, o_ref):
    o_ref[...] = (
        jnp.dot(agg_ref[...], wrel_ref[...],
                preferred_element_type=jnp.float32)
        + brel_ref[...]
        + jnp.dot(h_ref[...], wroot_ref[...],
                  preferred_element_type=jnp.float32))


def _tc_layer(agg, h, wrel, brel, wroot):
    return pl.pallas_call(
        _layer_body,
        grid=(GRID,),
        in_specs=[
            pl.BlockSpec((ROWB, D), lambda i: (i, 0)),
            pl.BlockSpec((ROWB, D), lambda i: (i, 0)),
            pl.BlockSpec((D, D), lambda i: (0, 0)),
            pl.BlockSpec((1, D), lambda i: (0, 0)),
            pl.BlockSpec((D, D), lambda i: (0, 0)),
        ],
        out_specs=pl.BlockSpec((ROWB, D), lambda i: (i, 0)),
        out_shape=jax.ShapeDtypeStruct((N, D), jnp.float32),
    )(agg, h, wrel, brel.reshape(1, D), wroot)


def _mlp_body(h_ref, wm0_ref, bm0_ref, wm1_ref, bm1_ref, o_ref):
    t = jnp.dot(h_ref[...], wm0_ref[...], preferred_element_type=jnp.float32)
    t = jnp.maximum(t + bm0_ref[...], 0.0)
    o = jnp.dot(t, wm1_ref[...], preferred_element_type=jnp.float32)
    o_ref[...] = jax.nn.sigmoid(o + bm1_ref[...])


def _tc_mlp(h, wm0, bm0, wm1, bm1):
    return pl.pallas_call(
        _mlp_body,
        grid=(GRID,),
        in_specs=[
            pl.BlockSpec((ROWB, D), lambda i: (i, 0)),
            pl.BlockSpec((D, D), lambda i: (0, 0)),
            pl.BlockSpec((1, D), lambda i: (0, 0)),
            pl.BlockSpec((D, 1), lambda i: (0, 0)),
            pl.BlockSpec((1, 1), lambda i: (0, 0)),
        ],
        out_specs=pl.BlockSpec((ROWB, 1), lambda i: (i, 0)),
        out_shape=jax.ShapeDtypeStruct((N, 1), jnp.float32),
    )(h, wm0, bm0.reshape(1, D), wm1, bm1.reshape(1, 1))


@functools.partial(
    pl.kernel,
    out_type=jax.ShapeDtypeStruct((N, D), jnp.float32),
    mesh=plsc.VectorSubcoreMesh(core_axis_name="c", subcore_axis_name="s"),
    compiler_params=pltpu.CompilerParams(needs_layout_passes=False),
    scratch_types=[
        pltpu.VMEM((K,), jnp.int32),
        pltpu.VMEM((K,), jnp.int32),
        pltpu.VMEM((STAGE,), jnp.int32),
        pltpu.VMEM((STAGE,), jnp.int32),
        pltpu.VMEM((2, CH), jnp.int32),
        pltpu.VMEM((2, CH), jnp.int32),
        pltpu.VMEM((2, CH, D), jnp.float32),
        pltpu.VMEM_SHARED((ACC_ROWS, D), jnp.float32),
        pltpu.VMEM_SHARED((E_PAD,), jnp.int32),
        pltpu.VMEM_SHARED((E_PAD,), jnp.int32),
        pltpu.SemaphoreType.DMA((2,)),
        pltpu.SemaphoreType.DMA,
    ],
)
def _sc_agg(h_hbm, src_hbm, dst_hbm, zeros_hbm, out_hbm,
            dchunk, schunk, stage_d, stage_s, flush_d, flush_s, rows,
            acc, dst_sp, src_sp, gsem, ssem):
    c = lax.axis_index("c")
    s = lax.axis_index("s")
    base = c * HALF0
    rpt = jnp.where(c == 0, R0, R1)
    lo = base + s * rpt
    hi = lo + rpt
    owned = jnp.where(c == 0, HALF0, NS * R1)
    dummy = owned + s
    lov = jnp.full((16,), lo, jnp.int32)
    hiv = jnp.full((16,), hi, jnp.int32)
    basev = jnp.full((16,), base, jnp.int32)
    dummyv = jnp.full((16,), dummy, jnp.int32)
    sentv = jnp.full((16,), SENTINEL, jnp.int32)

    @pl.when(s < 15)
    def _():
        pltpu.sync_copy(zeros_hbm.at[pl.ds(s * 320, 320)],
                        acc.at[pl.ds(s * 320, 320)])

    @pl.when(s == 15)
    def _():
        pltpu.sync_copy(zeros_hbm.at[pl.ds(4800, 224)],
                        acc.at[pl.ds(4800, 224)])

    pltpu.sync_copy(dst_hbm.at[pl.ds(s * EPT, EPT)],
                    dst_sp.at[pl.ds(s * EPT, EPT)])
    pltpu.sync_copy(src_hbm.at[pl.ds(s * EPT, EPT)],
                    src_sp.at[pl.ds(s * EPT, EPT)])
    plsc.subcore_barrier()

    def wait_gather(q):
        pltpu.make_async_copy(h_hbm.at[pl.ds(0, CH)], rows.at[q],
                              gsem.at[q]).wait()

    def wait_scatter():
        pltpu.make_async_copy(h_hbm.at[pl.ds(0, CH)], rows.at[0], ssem).wait()

    def flush_event(fo, nf):
        p = nf % 2

        @pl.when(nf >= 2)
        def _():
            wait_scatter()

        for kk in range(CH // 16):
            flush_d[p, pl.ds(kk * 16, 16)] = stage_d[pl.ds(fo + kk * 16, 16)]
            flush_s[p, pl.ds(kk * 16, 16)] = stage_s[pl.ds(fo + kk * 16, 16)]
        pltpu.async_copy(h_hbm.at[flush_s.at[p]], rows.at[p], gsem.at[p])

        @pl.when(nf >= 1)
        def _():
            q = (nf - 1) % 2
            wait_gather(q)
            pltpu.async_copy(rows.at[q], acc.at[flush_d.at[q]], ssem,
                             add=True)

    def scan_chunk(k, carry):
        ptr, nf = carry
        pltpu.sync_copy(dst_sp.at[pl.ds(k * K, K)], dchunk)
        pltpu.sync_copy(src_sp.at[pl.ds(k * K, K)], schunk)

        def scan_vec(i, ptr):
            dvec = dchunk[pl.ds(i * 16, 16)]
            svec = schunk[pl.ds(i * 16, 16)]
            fake = dvec >= sentv
            m = jnp.logical_or(
                jnp.logical_and(dvec >= lov, dvec < hiv), fake)
            cnt = jnp.sum(m.astype(jnp.int32))
            dl = jnp.where(fake, dummyv, dvec - basev)
            plsc.store_compressed(stage_d.at[pl.ds(ptr, 16)], dl, mask=m)
            plsc.store_compressed(stage_s.at[pl.ds(ptr, 16)], svec, mask=m)
            return ptr + cnt

        ptr = lax.fori_loop(0, K // 16, scan_vec, ptr, unroll=4)

        def flush_cond(c2):
            fo, nf = c2
            return fo + CH <= ptr

        def flush_step(c2):
            fo, nf = c2
            flush_event(fo, nf)
            return (fo + CH, nf + 1)

        fo, nf = lax.while_loop(flush_cond, flush_step, (jnp.int32(0), nf))

        for kk in range(CH // 16):
            v_d = stage_d[pl.ds(fo + kk * 16, 16)]
            v_s = stage_s[pl.ds(fo + kk * 16, 16)]
            stage_d[pl.ds(kk * 16, 16)] = v_d
            stage_s[pl.ds(kk * 16, 16)] = v_s
        return (ptr - fo, nf)

    _, nf = lax.fori_loop(0, NCHUNK, scan_chunk,
                          (jnp.int32(0), jnp.int32(0)))

    @pl.when(nf >= 2)
    def _():
        wait_scatter()

    @pl.when(nf >= 1)
    def _():
        q = (nf - 1) % 2
        wait_gather(q)
        pltpu.async_copy(rows.at[q], acc.at[flush_d.at[q]], ssem, add=True)
        wait_scatter()

    plsc.subcore_barrier()

    @pl.when(s < 15)
    def _():
        pltpu.sync_copy(acc.at[pl.ds(s * 320, 320)],
                        out_hbm.at[pl.ds(base + s * 320, 320)])

    @pl.when(jnp.logical_and(s == 15, c == 0))
    def _():
        pltpu.sync_copy(acc.at[pl.ds(4800, 208)],
                        out_hbm.at[pl.ds(4800, 208)])

    @pl.when(jnp.logical_and(s == 15, c == 1))
    def _():
        pltpu.sync_copy(acc.at[pl.ds(4800, 192)],
                        out_hbm.at[pl.ds(HALF0 + 4800, 192)])


def kernel(x, edge_index, Wrel0, brel0, Wroot0, Wrel1, brel1, Wroot1,
           Wrel2, brel2, Wroot2, Wm0, bm0, Wm1, bm1):
    src = edge_index[0].astype(jnp.int32)
    dst = edge_index[1].astype(jnp.int32)
    pad = E_PAD - E
    src_p = jnp.concatenate([src, jnp.zeros((pad,), jnp.int32)])
    dst_p = jnp.concatenate([dst, jnp.full((pad,), SENTINEL, jnp.int32)])
    zeros = jnp.zeros((ACC_ROWS, D), jnp.float32)

    h = x
    for wrel, brel, wroot in ((Wrel0, brel0, Wroot0),
                              (Wrel1, brel1, Wroot1),
                              (Wrel2, brel2, Wroot2)):
        agg = _sc_agg(h, src_p, dst_p, zeros)
        h = _tc_layer(agg, h, wrel, brel, wroot)
    return _tc_mlp(h, Wm0, bm0, Wm1, bm1)


# speedup vs baseline: 1.2271x; 1.0225x over previous
import functools

import jax
import jax.numpy as jnp
from jax import lax
from jax.experimental import pallas as pl
from jax.experimental.pallas import tpu as pltpu
from jax.experimental.pallas import tpu_sc as plsc

N = 10000
E = 320000
D = 128
ROWB = 2000
GRID = N // ROWB

NC, NS = 2, 16
K = 2048
NCHUNK = 157
E_PAD = K * NCHUNK
SENTINEL = 1 << 20

R0 = 313
R1 = 312
HALF0 = NS * R0
ACC_ROWS = 5024
EPT = E_PAD // NS
CH = 128
STAGE = 2304


def _layer_body(agg_ref, h_ref, wrel_ref, brel_ref, wroot_ref, o_ref):
    o_ref[...] = (
        jnp.dot(agg_ref[...], wrel_ref[...],
                preferred_element_type=jnp.float32)
        + brel_ref[...]
        + jnp.dot(h_ref[...], wroot_ref[...],
                  preferred_element_type=jnp.float32))


def _tc_layer(agg, h, wrel, brel, wroot):
    return pl.pallas_call(
        _layer_body,
        grid=(GRID,),
        in_specs=[
            pl.BlockSpec((ROWB, D), lambda i: (i, 0)),
            pl.BlockSpec((ROWB, D), lambda i: (i, 0)),
            pl.BlockSpec((D, D), lambda i: (0, 0)),
            pl.BlockSpec((1, D), lambda i: (0, 0)),
            pl.BlockSpec((D, D), lambda i: (0, 0)),
        ],
        out_specs=pl.BlockSpec((ROWB, D), lambda i: (i, 0)),
        out_shape=jax.ShapeDtypeStruct((N, D), jnp.float32),
    )(agg, h, wrel, brel.reshape(1, D), wroot)


def _mlp_body(h_ref, wm0_ref, bm0_ref, wm1_ref, bm1_ref, o_ref):
    t = jnp.dot(h_ref[...], wm0_ref[...], preferred_element_type=jnp.float32)
    t = jnp.maximum(t + bm0_ref[...], 0.0)
    o = jnp.dot(t, wm1_ref[...], preferred_element_type=jnp.float32)
    o_ref[...] = jax.nn.sigmoid(o + bm1_ref[...])


def _tc_mlp(h, wm0, bm0, wm1, bm1):
    return pl.pallas_call(
        _mlp_body,
        grid=(GRID,),
        in_specs=[
            pl.BlockSpec((ROWB, D), lambda i: (i, 0)),
            pl.BlockSpec((D, D), lambda i: (0, 0)),
            pl.BlockSpec((1, D), lambda i: (0, 0)),
            pl.BlockSpec((D, 1), lambda i: (0, 0)),
            pl.BlockSpec((1, 1), lambda i: (0, 0)),
        ],
        out_specs=pl.BlockSpec((ROWB, 1), lambda i: (i, 0)),
        out_shape=jax.ShapeDtypeStruct((N, 1), jnp.float32),
    )(h, wm0, bm0.reshape(1, D), wm1, bm1.reshape(1, 1))


@functools.partial(
    pl.kernel,
    out_type=jax.ShapeDtypeStruct((N, D), jnp.float32),
    mesh=plsc.VectorSubcoreMesh(core_axis_name="c", subcore_axis_name="s"),
    compiler_params=pltpu.CompilerParams(needs_layout_passes=False),
    scratch_types=[
        pltpu.VMEM((2, K), jnp.int32),
        pltpu.VMEM((2, K), jnp.int32),
        pltpu.VMEM((STAGE,), jnp.int32),
        pltpu.VMEM((STAGE,), jnp.int32),
        pltpu.VMEM((4, CH), jnp.int32),
        pltpu.VMEM((4, CH), jnp.int32),
        pltpu.VMEM((4, CH, D), jnp.float32),
        pltpu.VMEM_SHARED((ACC_ROWS, D), jnp.float32),
        pltpu.SemaphoreType.DMA((4,)),
        pltpu.SemaphoreType.DMA,
        pltpu.SemaphoreType.DMA((2,)),
    ],
)
def _sc_agg(h_hbm, src_hbm, dst_hbm, zeros_hbm, out_hbm,
            dchunk, schunk, stage_d, stage_s, flush_d, flush_s, rows,
            acc, gsem, ssem, csem):
    c = lax.axis_index("c")
    s = lax.axis_index("s")
    base = c * HALF0
    rpt = jnp.where(c == 0, R0, R1)
    lo = base + s * rpt
    hi = lo + rpt
    owned = jnp.where(c == 0, HALF0, NS * R1)
    dummy = owned + s
    lov = jnp.full((16,), lo, jnp.int32)
    hiv = jnp.full((16,), hi, jnp.int32)
    basev = jnp.full((16,), base, jnp.int32)
    dummyv = jnp.full((16,), dummy, jnp.int32)
    sentv = jnp.full((16,), SENTINEL, jnp.int32)

    @pl.when(s < 15)
    def _():
        pltpu.sync_copy(zeros_hbm.at[pl.ds(s * 320, 320)],
                        acc.at[pl.ds(s * 320, 320)])

    @pl.when(s == 15)
    def _():
        pltpu.sync_copy(zeros_hbm.at[pl.ds(4800, 224)],
                        acc.at[pl.ds(4800, 224)])

    plsc.subcore_barrier()

    def start_chunk_load(k):
        sl = k % 2
        pltpu.async_copy(dst_hbm.at[pl.ds(k * K, K)], dchunk.at[sl],
                         csem.at[sl])
        pltpu.async_copy(src_hbm.at[pl.ds(k * K, K)], schunk.at[sl],
                         csem.at[sl])

    def wait_chunk_load(k):
        sl = k % 2
        pltpu.make_async_copy(dst_hbm.at[pl.ds(0, K)], dchunk.at[sl],
                              csem.at[sl]).wait()
        pltpu.make_async_copy(src_hbm.at[pl.ds(0, K)], schunk.at[sl],
                              csem.at[sl]).wait()

    start_chunk_load(0)

    def wait_gather(q):
        pltpu.make_async_copy(h_hbm.at[pl.ds(0, CH)], rows.at[q],
                              gsem.at[q]).wait()

    def wait_scatter():
        pltpu.make_async_copy(h_hbm.at[pl.ds(0, CH)], rows.at[0], ssem).wait()

    def flush_event(fo, nf):
        p = nf % 4

        @pl.when(nf >= 4)
        def _():
            wait_scatter()

        for kk in range(CH // 16):
            flush_d[p, pl.ds(kk * 16, 16)] = stage_d[pl.ds(fo + kk * 16, 16)]
            flush_s[p, pl.ds(kk * 16, 16)] = stage_s[pl.ds(fo + kk * 16, 16)]
        pltpu.async_copy(h_hbm.at[flush_s.at[p]], rows.at[p], gsem.at[p])

        @pl.when(nf >= 1)
        def _():
            q = (nf - 1) % 4
            wait_gather(q)
            pltpu.async_copy(rows.at[q], acc.at[flush_d.at[q]], ssem,
                             add=True)

    def scan_chunk(k, carry):
        ptr, nf = carry
        wait_chunk_load(k)

        @pl.when(k + 1 < NCHUNK)
        def _():
            start_chunk_load(k + 1)

        sl = k % 2

        def scan_vec(i, ptr):
            dvec = dchunk[sl, pl.ds(i * 16, 16)]
            svec = schunk[sl, pl.ds(i * 16, 16)]
            fake = dvec >= sentv
            m = jnp.logical_or(
                jnp.logical_and(dvec >= lov, dvec < hiv), fake)
            cnt = jnp.sum(m.astype(jnp.int32))
            dl = jnp.where(fake, dummyv, dvec - basev)
            plsc.store_compressed(stage_d.at[pl.ds(ptr, 16)], dl, mask=m)
            plsc.store_compressed(stage_s.at[pl.ds(ptr, 16)], svec, mask=m)
            return ptr + cnt

        ptr = lax.fori_loop(0, K // 16, scan_vec, ptr, unroll=4)

        def flush_cond(c2):
            fo, nf = c2
            return fo + CH <= ptr

        def flush_step(c2):
            fo, nf = c2
            flush_event(fo, nf)
            return (fo + CH, nf + 1)

        fo, nf = lax.while_loop(flush_cond, flush_step, (jnp.int32(0), nf))

        for kk in range(CH // 16):
            v_d = stage_d[pl.ds(fo + kk * 16, 16)]
            v_s = stage_s[pl.ds(fo + kk * 16, 16)]
            stage_d[pl.ds(kk * 16, 16)] = v_d
            stage_s[pl.ds(kk * 16, 16)] = v_s
        return (ptr - fo, nf)

    _, nf = lax.fori_loop(0, NCHUNK, scan_chunk,
                          (jnp.int32(0), jnp.int32(0)))

    @pl.when(nf >= 1)
    def _():
        q = (nf - 1) % 4
        wait_gather(q)
        pltpu.async_copy(rows.at[q], acc.at[flush_d.at[q]], ssem, add=True)

    def drain(i, carry):
        wait_scatter()
        return carry

    lax.fori_loop(0, jnp.minimum(nf, 4), drain, jnp.int32(0))
    plsc.subcore_barrier()

    @pl.when(s < 15)
    def _():
        pltpu.sync_copy(acc.at[pl.ds(s * 320, 320)],
                        out_hbm.at[pl.ds(base + s * 320, 320)])

    @pl.when(jnp.logical_and(s == 15, c == 0))
    def _():
        pltpu.sync_copy(acc.at[pl.ds(4800, 208)],
                        out_hbm.at[pl.ds(4800, 208)])

    @pl.when(jnp.logical_and(s == 15, c == 1))
    def _():
        pltpu.sync_copy(acc.at[pl.ds(4800, 192)],
                        out_hbm.at[pl.ds(HALF0 + 4800, 192)])


def kernel(x, edge_index, Wrel0, brel0, Wroot0, Wrel1, brel1, Wroot1,
           Wrel2, brel2, Wroot2, Wm0, bm0, Wm1, bm1):
    src = edge_index[0].astype(jnp.int32)
    dst = edge_index[1].astype(jnp.int32)
    pad = E_PAD - E
    src_p = jnp.concatenate([src, jnp.zeros((pad,), jnp.int32)])
    dst_p = jnp.concatenate([dst, jnp.full((pad,), SENTINEL, jnp.int32)])
    zeros = jnp.zeros((ACC_ROWS, D), jnp.float32)

    h = x
    for wrel, brel, wroot in ((Wrel0, brel0, Wroot0),
                              (Wrel1, brel1, Wroot1),
                              (Wrel2, brel2, Wroot2)):
        agg = _sc_agg(h, src_p, dst_p, zeros)
        h = _tc_layer(agg, h, wrel, brel, wroot)
    return _tc_mlp(h, Wm0, bm0, Wm1, bm1)
